# baseline (speedup 1.0000x reference)
import functools

import jax
import jax.numpy as jnp
from jax.experimental import pallas as pl

_POS_MIN, _POS_MAX = -1.0, 1.0
_PERIOD = _POS_MAX - _POS_MIN


def _wrap(x):
    n = jnp.floor((x - _POS_MIN) / _PERIOD)
    xw = x - n * _PERIOD
    sign = 1.0 - 2.0 * jnp.mod(n, 2.0)
    return xw, sign


def _expand_mat(In, IO):
    Out = IO // In
    i_idx = jax.lax.broadcasted_iota(jnp.int32, (In, IO), 0)
    io_idx = jax.lax.broadcasted_iota(jnp.int32, (In, IO), 1)
    return (io_idx // Out == i_idx).astype(jnp.float32)


def _reduce_mat(IO, Out):
    io_idx = jax.lax.broadcasted_iota(jnp.int32, (IO, Out), 0)
    o_idx = jax.lax.broadcasted_iota(jnp.int32, (IO, Out), 1)
    return (io_idx % Out == o_idx).astype(jnp.float32)


def _pwl_flat(x, pos, val):
    In = x.shape[1]
    P, IO = pos.shape
    Out = IO // In
    xw, sign = _wrap(x)
    R = _expand_mat(In, IO)
    q = jnp.dot(xw, R, preferred_element_type=jnp.float32, precision=jax.lax.Precision.HIGHEST)
    s = jnp.dot(sign, R, preferred_element_type=jnp.float32, precision=jax.lax.Precision.HIGHEST)
    slopes = [
        (val[p + 1] - val[p]) / (pos[p + 1] - pos[p] + 1e-12)
        for p in range(P - 1)
    ]
    acc = val[0][None, :] + (q - pos[0][None, :]) * slopes[0][None, :]
    for p in range(1, P - 1):
        v = val[p][None, :] + (q - pos[p][None, :]) * slopes[p][None, :]
        acc = jnp.where(q >= pos[p][None, :], v, acc)
    E = _reduce_mat(IO, Out)
    return jnp.dot(acc * s, E, preferred_element_type=jnp.float32, precision=jax.lax.Precision.HIGHEST)


def _block_kernel(x_ref, pos1_ref, val1_ref, pos2_ref, val2_ref, o_ref):
    h = _pwl_flat(x_ref[...], pos1_ref[...], val1_ref[...])
    o_ref[...] = _pwl_flat(h, pos2_ref[...], val2_ref[...])


@functools.partial(jax.jit, static_argnames=("block_b",))
def _run(x, pos1_t, val1_t, pos2_t, val2_t, block_b=256):
    B, In = x.shape
    P, IO1 = pos1_t.shape
    IO2 = pos2_t.shape[1]
    O2 = IO2 // (IO1 // In)
    grid = (B // block_b,)
    return pl.pallas_call(
        _block_kernel,
        grid=grid,
        in_specs=[
            pl.BlockSpec((block_b, In), lambda j: (j, 0)),
            pl.BlockSpec((P, IO1), lambda j: (0, 0)),
            pl.BlockSpec((P, IO1), lambda j: (0, 0)),
            pl.BlockSpec((P, IO2), lambda j: (0, 0)),
            pl.BlockSpec((P, IO2), lambda j: (0, 0)),
        ],
        out_specs=pl.BlockSpec((block_b, O2), lambda j: (j, 0)),
        out_shape=jax.ShapeDtypeStruct((B, O2), x.dtype),
    )(x, pos1_t, val1_t, pos2_t, val2_t)


def kernel(x, pos1, val1, pos2, val2):
    def flat(t):
        In, Out, P = t.shape
        return jnp.transpose(t, (2, 0, 1)).reshape(P, In * Out)
    return _run(x, flat(pos1), flat(val1), flat(pos2), flat(val2))


# speedup vs baseline: 1.3904x; 1.3904x over previous
import functools

import jax
import jax.numpy as jnp
from jax.experimental import pallas as pl

_POS_MIN, _POS_MAX = -1.0, 1.0
_PERIOD = _POS_MAX - _POS_MIN


def _wrap(x):
    n = jnp.floor((x - _POS_MIN) / _PERIOD)
    xw = x - n * _PERIOD
    sign = 1.0 - 2.0 * jnp.mod(n, 2.0)
    return xw, sign


def _reduce_mat(IO, Out):
    In = IO // Out
    io_idx = jax.lax.broadcasted_iota(jnp.int32, (IO, Out), 0)
    o_idx = jax.lax.broadcasted_iota(jnp.int32, (IO, Out), 1)
    return (io_idx // In == o_idx).astype(jnp.float32)


def _pwl_flat(x, pos, val):
    In = x.shape[1]
    P, IO = pos.shape
    Out = IO // In
    xw, sign = _wrap(x)
    q = jnp.concatenate([xw] * Out, axis=1)
    s = jnp.concatenate([sign] * Out, axis=1)
    slopes = [
        (val[p + 1] - val[p]) / (pos[p + 1] - pos[p] + 1e-12)
        for p in range(P - 1)
    ]
    acc = val[0][None, :] + (q - pos[0][None, :]) * slopes[0][None, :]
    for p in range(1, P - 1):
        v = val[p][None, :] + (q - pos[p][None, :]) * slopes[p][None, :]
        acc = jnp.where(q >= pos[p][None, :], v, acc)
    E = _reduce_mat(IO, Out)
    return jnp.dot(acc * s, E, preferred_element_type=jnp.float32,
                   precision=jax.lax.Precision.HIGHEST)


def _block_kernel(x_ref, pos1_ref, val1_ref, pos2_ref, val2_ref, o_ref):
    h = _pwl_flat(x_ref[...], pos1_ref[...], val1_ref[...])
    o_ref[...] = _pwl_flat(h, pos2_ref[...], val2_ref[...])


@functools.partial(jax.jit, static_argnames=("block_b",))
def _run(x, pos1_t, val1_t, pos2_t, val2_t, block_b=256):
    B, In = x.shape
    P, IO1 = pos1_t.shape
    IO2 = pos2_t.shape[1]
    O2 = IO2 // In
    grid = (B // block_b,)
    return pl.pallas_call(
        _block_kernel,
        grid=grid,
        in_specs=[
            pl.BlockSpec((block_b, In), lambda j: (j, 0)),
            pl.BlockSpec((P, IO1), lambda j: (0, 0)),
            pl.BlockSpec((P, IO1), lambda j: (0, 0)),
            pl.BlockSpec((P, IO2), lambda j: (0, 0)),
            pl.BlockSpec((P, IO2), lambda j: (0, 0)),
        ],
        out_specs=pl.BlockSpec((block_b, O2), lambda j: (j, 0)),
        out_shape=jax.ShapeDtypeStruct((B, O2), x.dtype),
    )(x, pos1_t, val1_t, pos2_t, val2_t)


def kernel(x, pos1, val1, pos2, val2):
    def flat(t):
        In, Out, P = t.shape
        return jnp.transpose(t, (2, 1, 0)).reshape(P, Out * In)
    return _run(x, flat(pos1), flat(val1), flat(pos2), flat(val2))


# speedup vs baseline: 1.3905x; 1.0000x over previous
import functools

import jax
import jax.numpy as jnp
from jax.experimental import pallas as pl
from jax.experimental.pallas import tpu as pltpu

_POS_MIN, _POS_MAX = -1.0, 1.0
_PERIOD = _POS_MAX - _POS_MIN


def _wrap(x):
    n = jnp.floor((x - _POS_MIN) / _PERIOD)
    xw = x - n * _PERIOD
    sign = 1.0 - 2.0 * jnp.mod(n, 2.0)
    return xw, sign


def _reduce_mat(IO, Out):
    In = IO // Out
    io_idx = jax.lax.broadcasted_iota(jnp.int32, (IO, Out), 0)
    o_idx = jax.lax.broadcasted_iota(jnp.int32, (IO, Out), 1)
    return (io_idx // In == o_idx).astype(jnp.float32)


def _pwl_flat(x, pos, val):
    In = x.shape[1]
    P, IO = pos.shape
    Out = IO // In
    xw, sign = _wrap(x)
    q = jnp.concatenate([xw] * Out, axis=1)
    s = jnp.concatenate([sign] * Out, axis=1)
    slopes = [
        (val[p + 1] - val[p]) / (pos[p + 1] - pos[p] + 1e-12)
        for p in range(P - 1)
    ]
    acc = val[0][None, :] + (q - pos[0][None, :]) * slopes[0][None, :]
    for p in range(1, P - 1):
        v = val[p][None, :] + (q - pos[p][None, :]) * slopes[p][None, :]
        acc = jnp.where(q >= pos[p][None, :], v, acc)
    E = _reduce_mat(IO, Out)
    return jnp.dot(acc * s, E, preferred_element_type=jnp.float32,
                   precision=jax.lax.Precision.HIGHEST)


def _block_kernel(x_ref, pos1_ref, val1_ref, pos2_ref, val2_ref, o_ref):
    h = _pwl_flat(x_ref[...], pos1_ref[...], val1_ref[...])
    o_ref[...] = _pwl_flat(h, pos2_ref[...], val2_ref[...])


@functools.partial(jax.jit, static_argnames=("block_b",))
def _run(x, pos1_t, val1_t, pos2_t, val2_t, block_b=256):
    B, In = x.shape
    P, IO1 = pos1_t.shape
    IO2 = pos2_t.shape[1]
    O2 = IO2 // In
    grid = (B // block_b,)
    return pl.pallas_call(
        _block_kernel,
        grid=grid,
        in_specs=[
            pl.BlockSpec((block_b, In), lambda j: (j, 0)),
            pl.BlockSpec((P, IO1), lambda j: (0, 0)),
            pl.BlockSpec((P, IO1), lambda j: (0, 0)),
            pl.BlockSpec((P, IO2), lambda j: (0, 0)),
            pl.BlockSpec((P, IO2), lambda j: (0, 0)),
        ],
        out_specs=pl.BlockSpec((block_b, O2), lambda j: (j, 0)),
        out_shape=jax.ShapeDtypeStruct((B, O2), x.dtype),
        compiler_params=pltpu.CompilerParams(
            dimension_semantics=("parallel",)),
    )(x, pos1_t, val1_t, pos2_t, val2_t)


def kernel(x, pos1, val1, pos2, val2):
    def flat(t):
        In, Out, P = t.shape
        return jnp.transpose(t, (2, 1, 0)).reshape(P, Out * In)
    return _run(x, flat(pos1), flat(val1), flat(pos2), flat(val2))


# speedup vs baseline: 1.6064x; 1.1553x over previous
import functools

import jax
import jax.numpy as jnp
from jax.experimental import pallas as pl
from jax.experimental.pallas import tpu as pltpu

_POS_MIN, _POS_MAX = -1.0, 1.0
_PERIOD = _POS_MAX - _POS_MIN


def _wrap(x):
    n = jnp.floor((x - _POS_MIN) / _PERIOD)
    xw = x - n * _PERIOD
    sign = 1.0 - 2.0 * jnp.mod(n, 2.0)
    return xw, sign


def _reduce_mat(IO, Out):
    In = IO // Out
    io_idx = jax.lax.broadcasted_iota(jnp.int32, (IO, Out), 0)
    o_idx = jax.lax.broadcasted_iota(jnp.int32, (IO, Out), 1)
    return (io_idx // In == o_idx).astype(jnp.float32)


def _pwl_flat(x, pos, val):
    In = x.shape[1]
    P, IO = pos.shape
    Out = IO // In
    xw, sign = _wrap(x)
    q = jnp.concatenate([xw] * Out, axis=1)
    s = jnp.concatenate([sign] * Out, axis=1)
    slopes = [
        (val[p + 1] - val[p]) / (pos[p + 1] - pos[p] + 1e-12)
        for p in range(P - 1)
    ]
    u_prev = jnp.minimum(q, pos[1][None, :])
    acc = val[0][None, :] + slopes[0][None, :] * (u_prev - pos[0][None, :])
    for p in range(1, P - 2):
        u = jnp.minimum(q, pos[p + 1][None, :])
        acc = acc + slopes[p][None, :] * (u - u_prev)
        u_prev = u
    acc = acc + slopes[P - 2][None, :] * (q - u_prev)
    E = _reduce_mat(IO, Out)
    return jnp.dot(acc * s, E, preferred_element_type=jnp.float32,
                   precision=jax.lax.Precision.HIGHEST)


def _block_kernel(x_ref, pos1_ref, val1_ref, pos2_ref, val2_ref, o_ref):
    h = _pwl_flat(x_ref[...], pos1_ref[...], val1_ref[...])
    o_ref[...] = _pwl_flat(h, pos2_ref[...], val2_ref[...])


@functools.partial(jax.jit, static_argnames=("block_b",))
def _run(x, pos1_t, val1_t, pos2_t, val2_t, block_b=256):
    B, In = x.shape
    P, IO1 = pos1_t.shape
    IO2 = pos2_t.shape[1]
    O2 = IO2 // In
    grid = (B // block_b,)
    return pl.pallas_call(
        _block_kernel,
        grid=grid,
        in_specs=[
            pl.BlockSpec((block_b, In), lambda j: (j, 0)),
            pl.BlockSpec((P, IO1), lambda j: (0, 0)),
            pl.BlockSpec((P, IO1), lambda j: (0, 0)),
            pl.BlockSpec((P, IO2), lambda j: (0, 0)),
            pl.BlockSpec((P, IO2), lambda j: (0, 0)),
        ],
        out_specs=pl.BlockSpec((block_b, O2), lambda j: (j, 0)),
        out_shape=jax.ShapeDtypeStruct((B, O2), x.dtype),
        compiler_params=pltpu.CompilerParams(
            dimension_semantics=("parallel",)),
    )(x, pos1_t, val1_t, pos2_t, val2_t)


def kernel(x, pos1, val1, pos2, val2):
    def flat(t):
        In, Out, P = t.shape
        return jnp.transpose(t, (2, 1, 0)).reshape(P, Out * In)
    return _run(x, flat(pos1), flat(val1), flat(pos2), flat(val2))


# speedup vs baseline: 1.6816x; 1.0468x over previous
import functools

import jax
import jax.numpy as jnp
from jax.experimental import pallas as pl
from jax.experimental.pallas import tpu as pltpu

_POS_MIN, _POS_MAX = -1.0, 1.0
_PERIOD = _POS_MAX - _POS_MIN


def _wrap(x):
    n = jnp.floor((x - _POS_MIN) / _PERIOD)
    xw = x - n * _PERIOD
    sign = 1.0 - 2.0 * jnp.mod(n, 2.0)
    return xw, sign


def _reduce_mat(IO, Out):
    In = IO // Out
    io_idx = jax.lax.broadcasted_iota(jnp.int32, (IO, Out), 0)
    o_idx = jax.lax.broadcasted_iota(jnp.int32, (IO, Out), 1)
    return (io_idx // In == o_idx).astype(jnp.float32)


def _pwl_flat(x, pos, val):
    In = x.shape[1]
    P, IO = pos.shape
    Out = IO // In
    xw, sign = _wrap(x)
    q = jnp.concatenate([xw] * Out, axis=1)
    s = jnp.concatenate([sign] * Out, axis=1)
    slopes = [
        (val[p + 1] - val[p]) / (pos[p + 1] - pos[p] + 1e-12)
        for p in range(P - 1)
    ]
    u_prev = jnp.minimum(q, pos[1][None, :])
    acc = val[0][None, :] + slopes[0][None, :] * (u_prev - pos[0][None, :])
    for p in range(1, P - 2):
        u = jnp.minimum(q, pos[p + 1][None, :])
        acc = acc + slopes[p][None, :] * (u - u_prev)
        u_prev = u
    acc = acc + slopes[P - 2][None, :] * (q - u_prev)
    E = _reduce_mat(IO, Out)
    return jnp.dot(acc * s, E, preferred_element_type=jnp.float32,
                   precision=jax.lax.Precision.HIGHEST)


def _block_kernel(x_ref, pos1_ref, val1_ref, pos2_ref, val2_ref, o_ref):
    h = _pwl_flat(x_ref[...], pos1_ref[...], val1_ref[...])
    o_ref[...] = _pwl_flat(h, pos2_ref[...], val2_ref[...])


@functools.partial(jax.jit, static_argnames=("block_b",))
def _run(x, pos1_t, val1_t, pos2_t, val2_t, block_b=512):
    B, In = x.shape
    P, IO1 = pos1_t.shape
    IO2 = pos2_t.shape[1]
    O2 = IO2 // In
    grid = (B // block_b,)
    return pl.pallas_call(
        _block_kernel,
        grid=grid,
        in_specs=[
            pl.BlockSpec((block_b, In), lambda j: (j, 0)),
            pl.BlockSpec((P, IO1), lambda j: (0, 0)),
            pl.BlockSpec((P, IO1), lambda j: (0, 0)),
            pl.BlockSpec((P, IO2), lambda j: (0, 0)),
            pl.BlockSpec((P, IO2), lambda j: (0, 0)),
        ],
        out_specs=pl.BlockSpec((block_b, O2), lambda j: (j, 0)),
        out_shape=jax.ShapeDtypeStruct((B, O2), x.dtype),
        compiler_params=pltpu.CompilerParams(
            dimension_semantics=("parallel",)),
    )(x, pos1_t, val1_t, pos2_t, val2_t)


def kernel(x, pos1, val1, pos2, val2):
    def flat(t):
        In, Out, P = t.shape
        return jnp.transpose(t, (2, 1, 0)).reshape(P, Out * In)
    return _run(x, flat(pos1), flat(val1), flat(pos2), flat(val2))
